# baseline (speedup 1.0000x reference)
import functools

import jax
import jax.numpy as jnp
from jax import lax
from jax.experimental import pallas as pl
from jax.experimental.pallas import tpu as pltpu
from jax.experimental.pallas import tpu_sc as plsc

B_ = 16384
DIM_ = 64
NEG_ = 20
HALF_ = 500_000
NC_ = 2
NS_ = 16
NW_ = NC_ * NS_
BPW_ = B_ // NW_
C_ = 32
NCHUNK_ = BPW_ // C_
NIDX_ROWS_ = C_ * NEG_ // 128
TROWS_ = BPW_ // 128


def _sc_dots(tgt, ctx, negf, emb2):
    mesh = plsc.VectorSubcoreMesh(core_axis_name="c", subcore_axis_name="s")

    @functools.partial(
        pl.kernel,
        mesh=mesh,
        compiler_params=pltpu.CompilerParams(
            needs_layout_passes=False, use_tc_tiling_on_sc=True),
        out_type=(jax.ShapeDtypeStruct((B_,), jnp.float32),
                  jax.ShapeDtypeStruct((B_,), jnp.float32)),
        scratch_types=[
            pltpu.VMEM((TROWS_, 128), jnp.int32),
            pltpu.VMEM((TROWS_, 128), jnp.int32),
            pltpu.VMEM((TROWS_, 128), jnp.int32),
            pltpu.VMEM((TROWS_, 128), jnp.int32),
            pltpu.VMEM((NCHUNK_ * NIDX_ROWS_, 128), jnp.int32),
            pltpu.VMEM((NCHUNK_ * NIDX_ROWS_, 128), jnp.int32),
            pltpu.VMEM((C_, 128), jnp.float32),
            pltpu.VMEM((C_, 128), jnp.float32),
            pltpu.VMEM((C_ * NEG_, 128), jnp.float32),
            pltpu.VMEM((C_,), jnp.float32),
            pltpu.VMEM((C_,), jnp.float32),
            pltpu.SemaphoreType.DMA,
            pltpu.SemaphoreType.DMA,
        ],
    )
    def run(tgt_hbm, ctx_hbm, negf_hbm, emb_hbm, dpos_hbm, dneg_hbm,
            tidx, cidx, trow, crow, nrow, ncol,
            vrows, urows, nrows, dposv, dnegv, semvu, semn):
        wid = lax.axis_index("s") * NC_ + lax.axis_index("c")
        base = wid * BPW_
        pltpu.sync_copy(tgt_hbm.at[wid], tidx)
        pltpu.sync_copy(ctx_hbm.at[wid], cidx)
        pltpu.sync_copy(negf_hbm.at[wid], nrow)

        half = jnp.full((16,), HALF_, jnp.int32)
        zero = jnp.zeros((16,), jnp.int32)
        c64 = jnp.full((16,), 64, jnp.int32)
        for r in range(TROWS_):
            for h in range(8):
                s = pl.ds(h * 16, 16)
                t = tidx[r, s]
                c = cidx[r, s]
                trow[r, s] = t - jnp.where(t >= half, half, zero)
                crow[r, s] = c - jnp.where(c >= half, half, zero)

        def remap_body(r, carry):
            for h in range(8):
                s = pl.ds(h * 16, 16)
                x = nrow[r, s]
                ge = x >= half
                ncol[r, s] = jnp.where(ge, c64, zero)
                nrow[r, s] = x - jnp.where(ge, half, zero)
            return carry

        lax.fori_loop(0, NCHUNK_ * NIDX_ROWS_, remap_body, 0)

        def vu_descs(ci):
            tr = trow.at[ci // 4, pl.ds((ci % 4) * C_, C_)]
            cr = crow.at[ci // 4, pl.ds((ci % 4) * C_, C_)]
            return [(emb_hbm.at[tr], vrows),
                    (emb_hbm.at[cr], urows)]

        def neg_descs(ci):
            return [(emb_hbm.at[nrow.at[ci * NIDX_ROWS_ + j]],
                     nrows.at[pl.ds(j * 128, 128)])
                    for j in range(NIDX_ROWS_)]

        def issue_vu(ci):
            for src, dst in vu_descs(ci):
                pltpu.async_copy(src, dst, semvu)

        def drain_vu(ci):
            for src, dst in vu_descs(ci):
                pltpu.make_async_copy(src, dst, semvu).wait()

        def issue_neg(ci):
            for src, dst in neg_descs(ci):
                pltpu.async_copy(src, dst, semn)

        def drain_neg(ci):
            for src, dst in neg_descs(ci):
                pltpu.make_async_copy(src, dst, semn).wait()

        lanes = lax.iota(jnp.int32, 16)

        def step(ci):
            start = base + ci * C_
            drain_neg(ci)
            drain_vu(ci)
            for g in range(C_ // 16):
                bidx = lanes + g * 16
                tcol = jnp.where(
                    tidx[ci // 4, pl.ds((ci % 4) * C_ + g * 16, 16)]
                    >= half, c64, zero)
                ccol = jnp.where(
                    cidx[ci // 4, pl.ds((ci % 4) * C_ + g * 16, 16)]
                    >= half, c64, zero)
                nbase = []
                for n in range(NEG_):
                    flat = bidx * NEG_ + n
                    nc = plsc.load_gather(
                        ncol,
                        [jax.lax.shift_right_logical(flat, 7)
                         + ci * NIDX_ROWS_,
                         flat & 127])
                    nbase.append((flat, nc))
                pacc = jnp.zeros((16,), jnp.float32)
                nacc = jnp.zeros((16,), jnp.float32)
                for d in range(DIM_):
                    dd = jnp.full((16,), d, jnp.int32)
                    vT = plsc.load_gather(vrows, [bidx, tcol + dd])
                    uT = plsc.load_gather(urows, [bidx, ccol + dd])
                    pacc = pacc + vT * uT
                    for n in range(NEG_):
                        fl, nc = nbase[n]
                        nT = plsc.load_gather(nrows, [fl, nc + dd])
                        nacc = nacc + vT * nT
                dposv[pl.ds(g * 16, 16)] = pacc
                dnegv[pl.ds(g * 16, 16)] = nacc

            @pl.when(ci + 1 < NCHUNK_)
            def _():
                issue_neg(ci + 1)
                issue_vu(ci + 1)

            pltpu.sync_copy(dposv, dpos_hbm.at[pl.ds(start, C_)])
            pltpu.sync_copy(dnegv, dneg_hbm.at[pl.ds(start, C_)])

        issue_vu(0)
        issue_neg(0)
        lax.fori_loop(0, NCHUNK_, lambda ci, c: (step(ci), c)[1], 0)

    return run(tgt, ctx, negf, emb2)


def _loss_body(dp_ref, dn_ref, o_ref):
    dp = dp_ref[...]
    dn = dn_ref[...]
    ls = (jnp.minimum(dp, 0.0) - jnp.log1p(jnp.exp(-jnp.abs(dp)))
          + jnp.minimum(-dn, 0.0) - jnp.log1p(jnp.exp(-jnp.abs(dn))))
    o_ref[0, 0] = -jnp.sum(ls) / B_


def _loss_tc(dpos, dneg):
    return pl.pallas_call(
        _loss_body,
        out_shape=jax.ShapeDtypeStruct((1, 1), jnp.float32),
        out_specs=pl.BlockSpec(memory_space=pltpu.SMEM),
    )(dpos, dneg)


def kernel(target_input, context, neg, emb):
    tgt = target_input.astype(jnp.int32).reshape(NW_, TROWS_, 128)
    ctx = context.astype(jnp.int32).reshape(NW_, TROWS_, 128)
    negf = neg.astype(jnp.int32).reshape(NW_, NCHUNK_ * NIDX_ROWS_, 128)
    emb2 = jnp.concatenate([emb[:HALF_], emb[HALF_:]], axis=1)
    dpos, dneg = _sc_dots(tgt, ctx, negf, emb2)
    out = _loss_tc(dpos.reshape(128, 128), dneg.reshape(128, 128))
    return out[0, 0]


# speedup vs baseline: 1.7621x; 1.7621x over previous
import functools

import jax
import jax.numpy as jnp
from jax import lax
from jax.experimental import pallas as pl
from jax.experimental.pallas import tpu as pltpu
from jax.experimental.pallas import tpu_sc as plsc

B_ = 16384
DIM_ = 64
NEG_ = 20
NC_ = 2
NS_ = 16
NW_ = NC_ * NS_
BPW_ = B_ // NW_
C_ = 32
NCHUNK_ = BPW_ // C_
NIDX_ROWS_ = C_ * NEG_ // 128


def _sc_dots(tgt, ctx, negf, emb):
    mesh = plsc.VectorSubcoreMesh(core_axis_name="c", subcore_axis_name="s")

    @functools.partial(
        pl.kernel,
        mesh=mesh,
        compiler_params=pltpu.CompilerParams(
            needs_layout_passes=False, use_tc_tiling_on_sc=False),
        out_type=(jax.ShapeDtypeStruct((B_,), jnp.float32),
                  jax.ShapeDtypeStruct((B_,), jnp.float32)),
        scratch_types=[
            pltpu.VMEM((NCHUNK_, C_), jnp.int32),
            pltpu.VMEM((NCHUNK_, C_), jnp.int32),
            pltpu.VMEM((NCHUNK_ * NIDX_ROWS_, 128), jnp.int32),
            pltpu.VMEM((2, C_, DIM_), jnp.float32),
            pltpu.VMEM((2, C_, DIM_), jnp.float32),
            pltpu.VMEM((2, C_ * NEG_, DIM_), jnp.float32),
            pltpu.VMEM((C_, DIM_), jnp.float32),
            pltpu.VMEM((C_,), jnp.float32),
            pltpu.VMEM((C_,), jnp.float32),
            pltpu.SemaphoreType.DMA,
            pltpu.SemaphoreType.DMA,
            pltpu.SemaphoreType.DMA,
            pltpu.SemaphoreType.DMA,
        ],
    )
    def run(tgt_hbm, ctx_hbm, negf_hbm, emb_hbm, dpos_hbm, dneg_hbm,
            tidx, cidx, nidx, vrows, urows, nrows, nsumv, dposv, dnegv,
            sem0, sem1, semn0, semn1):
        wid = lax.axis_index("s") * NC_ + lax.axis_index("c")
        base = wid * BPW_
        pltpu.sync_copy(tgt_hbm.at[wid], tidx)
        pltpu.sync_copy(ctx_hbm.at[wid], cidx)
        pltpu.sync_copy(negf_hbm.at[wid], nidx)

        sems_vu = (sem0, sem1)
        sems_neg = (semn0, semn1)

        def neg_descs(ci, s):
            return [(emb_hbm.at[nidx.at[ci * NIDX_ROWS_ + j]],
                     nrows.at[s].at[pl.ds(j * 128, 128)])
                    for j in range(NIDX_ROWS_)]

        def vu_descs(ci, s):
            return [(emb_hbm.at[tidx.at[ci]], vrows.at[s]),
                    (emb_hbm.at[cidx.at[ci]], urows.at[s])]

        def issue(ci, s):
            for src, dst in neg_descs(ci, s):
                pltpu.async_copy(src, dst, sems_neg[s])
            for src, dst in vu_descs(ci, s):
                pltpu.async_copy(src, dst, sems_vu[s])

        def drain_neg(ci, s):
            for src, dst in neg_descs(ci, s):
                pltpu.make_async_copy(src, dst, sems_neg[s]).wait()

        def drain_vu(ci, s):
            for src, dst in vu_descs(ci, s):
                pltpu.make_async_copy(src, dst, sems_vu[s]).wait()

        lanes = lax.iota(jnp.int32, 16)

        def compute(ci, s):
            start = base + ci * C_
            drain_neg(ci, s)
            def sum_body(b, carry):
                sq = [nrows[s, b * NEG_, pl.ds(q * 16, 16)]
                      for q in range(4)]
                for n in range(1, NEG_):
                    for q in range(4):
                        sq[q] = sq[q] + nrows[s, b * NEG_ + n,
                                              pl.ds(q * 16, 16)]
                for q in range(4):
                    nsumv[b, pl.ds(q * 16, 16)] = sq[q]
                return carry

            lax.fori_loop(0, C_, sum_body, 0)
            drain_vu(ci, s)
            for g in range(C_ // 16):
                bidx = lanes + g * 16
                pacc = jnp.zeros((16,), jnp.float32)
                nacc = jnp.zeros((16,), jnp.float32)
                for d in range(DIM_):
                    dd = jnp.full((16,), d, jnp.int32)
                    vT = plsc.load_gather(vrows.at[s], [bidx, dd])
                    uT = plsc.load_gather(urows.at[s], [bidx, dd])
                    nT = plsc.load_gather(nsumv, [bidx, dd])
                    pacc = pacc + vT * uT
                    nacc = nacc + vT * nT
                dposv[pl.ds(g * 16, 16)] = pacc
                dnegv[pl.ds(g * 16, 16)] = nacc
            pltpu.sync_copy(dposv, dpos_hbm.at[pl.ds(start, C_)])
            pltpu.sync_copy(dnegv, dneg_hbm.at[pl.ds(start, C_)])

        issue(0, 0)
        issue(1, 1)

        def pipe_body(g, carry):
            c0 = g * 2
            compute(c0, 0)

            @pl.when(g < NCHUNK_ // 2 - 1)
            def _():
                issue(c0 + 2, 0)

            compute(c0 + 1, 1)

            @pl.when(g < NCHUNK_ // 2 - 1)
            def _():
                issue(c0 + 3, 1)

            return carry

        lax.fori_loop(0, NCHUNK_ // 2, pipe_body, 0)

    return run(tgt, ctx, negf, emb)


def _loss_body(dp_ref, dn_ref, o_ref):
    dp = dp_ref[...]
    dn = dn_ref[...]
    ls = (jnp.minimum(dp, 0.0) - jnp.log1p(jnp.exp(-jnp.abs(dp)))
          + jnp.minimum(-dn, 0.0) - jnp.log1p(jnp.exp(-jnp.abs(dn))))
    o_ref[0, 0] = -jnp.sum(ls) / B_


def _loss_tc(dpos, dneg):
    return pl.pallas_call(
        _loss_body,
        out_shape=jax.ShapeDtypeStruct((1, 1), jnp.float32),
        out_specs=pl.BlockSpec(memory_space=pltpu.SMEM),
    )(dpos, dneg)


def kernel(target_input, context, neg, emb):
    tgt = target_input.astype(jnp.int32).reshape(NW_, NCHUNK_, C_)
    ctx = context.astype(jnp.int32).reshape(NW_, NCHUNK_, C_)
    negf = neg.astype(jnp.int32).reshape(NW_, NCHUNK_ * NIDX_ROWS_, 128)
    dpos, dneg = _sc_dots(tgt, ctx, negf, emb)
    out = _loss_tc(dpos.reshape(128, 128), dneg.reshape(128, 128))
    return out[0, 0]


# speedup vs baseline: 1.7661x; 1.0023x over previous
import functools

import jax
import jax.numpy as jnp
from jax import lax
from jax.experimental import pallas as pl
from jax.experimental.pallas import tpu as pltpu
from jax.experimental.pallas import tpu_sc as plsc

B_ = 16384
DIM_ = 64
NEG_ = 20
NC_ = 2
NS_ = 16
NW_ = NC_ * NS_
BPW_ = B_ // NW_
C_ = 32
NCHUNK_ = BPW_ // C_
NIDX_ROWS_ = C_ * NEG_ // 128


def _sc_dots(tgt, ctx, negf, emb):
    mesh = plsc.VectorSubcoreMesh(core_axis_name="c", subcore_axis_name="s")

    @functools.partial(
        pl.kernel,
        mesh=mesh,
        compiler_params=pltpu.CompilerParams(
            needs_layout_passes=False, use_tc_tiling_on_sc=False),
        out_type=(jax.ShapeDtypeStruct((B_,), jnp.float32),
                  jax.ShapeDtypeStruct((B_,), jnp.float32)),
        scratch_types=[
            pltpu.VMEM((NCHUNK_, C_), jnp.int32),
            pltpu.VMEM((NCHUNK_, C_), jnp.int32),
            pltpu.VMEM((NCHUNK_ * NIDX_ROWS_, 128), jnp.int32),
            pltpu.VMEM((2, C_, DIM_), jnp.float32),
            pltpu.VMEM((2, C_, DIM_), jnp.float32),
            pltpu.VMEM((2, C_ * NEG_, DIM_), jnp.float32),
            pltpu.VMEM((C_, DIM_), jnp.float32),
            pltpu.VMEM((C_,), jnp.float32),
            pltpu.VMEM((C_,), jnp.float32),
            pltpu.SemaphoreType.DMA,
            pltpu.SemaphoreType.DMA,
        ],
    )
    def run(tgt_hbm, ctx_hbm, negf_hbm, emb_hbm, dpos_hbm, dneg_hbm,
            tidx, cidx, nidx, vrows, urows, nrows, nsumv, dposv, dnegv,
            sem0, sem1):
        wid = lax.axis_index("s") * NC_ + lax.axis_index("c")
        base = wid * BPW_
        pltpu.sync_copy(tgt_hbm.at[wid], tidx)
        pltpu.sync_copy(ctx_hbm.at[wid], cidx)
        pltpu.sync_copy(negf_hbm.at[wid], nidx)

        sems = (sem0, sem1)

        def descs(ci, s):
            out = [(emb_hbm.at[tidx.at[ci]], vrows.at[s]),
                   (emb_hbm.at[cidx.at[ci]], urows.at[s])]
            for j in range(NIDX_ROWS_):
                out.append((emb_hbm.at[nidx.at[ci * NIDX_ROWS_ + j]],
                            nrows.at[s].at[pl.ds(j * 128, 128)]))
            return out

        def issue(ci, s):
            for src, dst in descs(ci, s):
                pltpu.async_copy(src, dst, sems[s])

        def drain(ci, s):
            for src, dst in descs(ci, s):
                pltpu.make_async_copy(src, dst, sems[s]).wait()

        lanes = lax.iota(jnp.int32, 16)

        def compute(ci, s):
            start = base + ci * C_
            def sum_body(b, carry):
                sq = [nrows[s, b * NEG_, pl.ds(q * 16, 16)]
                      for q in range(4)]
                for n in range(1, NEG_):
                    for q in range(4):
                        sq[q] = sq[q] + nrows[s, b * NEG_ + n,
                                              pl.ds(q * 16, 16)]
                for q in range(4):
                    nsumv[b, pl.ds(q * 16, 16)] = sq[q]
                return carry

            lax.fori_loop(0, C_, sum_body, 0)
            for g in range(C_ // 16):
                bidx = lanes + g * 16
                pacc = jnp.zeros((16,), jnp.float32)
                nacc = jnp.zeros((16,), jnp.float32)
                for d in range(DIM_):
                    dd = jnp.full((16,), d, jnp.int32)
                    vT = plsc.load_gather(vrows.at[s], [bidx, dd])
                    uT = plsc.load_gather(urows.at[s], [bidx, dd])
                    nT = plsc.load_gather(nsumv, [bidx, dd])
                    pacc = pacc + vT * uT
                    nacc = nacc + vT * nT
                dposv[pl.ds(g * 16, 16)] = pacc
                dnegv[pl.ds(g * 16, 16)] = nacc
            pltpu.sync_copy(dposv, dpos_hbm.at[pl.ds(start, C_)])
            pltpu.sync_copy(dnegv, dneg_hbm.at[pl.ds(start, C_)])

        issue(0, 0)
        issue(1, 1)

        def pipe_body(g, carry):
            c0 = g * 2
            drain(c0, 0)
            compute(c0, 0)

            @pl.when(g < NCHUNK_ // 2 - 1)
            def _():
                issue(c0 + 2, 0)

            drain(c0 + 1, 1)
            compute(c0 + 1, 1)

            @pl.when(g < NCHUNK_ // 2 - 1)
            def _():
                issue(c0 + 3, 1)

            return carry

        lax.fori_loop(0, NCHUNK_ // 2, pipe_body, 0)

    return run(tgt, ctx, negf, emb)


def _loss_body(dp_ref, dn_ref, o_ref):
    dp = dp_ref[...]
    dn = dn_ref[...]
    ls = (jnp.minimum(dp, 0.0) - jnp.log1p(jnp.exp(-jnp.abs(dp)))
          + jnp.minimum(-dn, 0.0) - jnp.log1p(jnp.exp(-jnp.abs(dn))))
    o_ref[0, 0] = -jnp.sum(ls) / B_


def _loss_tc(dpos, dneg):
    return pl.pallas_call(
        _loss_body,
        out_shape=jax.ShapeDtypeStruct((1, 1), jnp.float32),
        out_specs=pl.BlockSpec(memory_space=pltpu.SMEM),
    )(dpos, dneg)


def kernel(target_input, context, neg, emb):
    tgt = target_input.astype(jnp.int32).reshape(NW_, NCHUNK_, C_)
    ctx = context.astype(jnp.int32).reshape(NW_, NCHUNK_, C_)
    negf = neg.astype(jnp.int32).reshape(NW_, NCHUNK_ * NIDX_ROWS_, 128)
    dpos, dneg = _sc_dots(tgt, ctx, negf, emb)
    out = _loss_tc(dpos.reshape(128, 128), dneg.reshape(128, 128))
    return out[0, 0]
